# baseline (speedup 1.0000x reference)
import functools

import jax
import jax.numpy as jnp
from jax import lax
from jax.experimental import pallas as pl
from jax.experimental.pallas import tpu as pltpu
from jax.experimental.pallas import tpu_sc as plsc

D = 128
SEQ = 200
BATCH = 4096
LANES = 16
NWORKERS = 32
CHUNKS = BATCH // NWORKERS
NBUF = 3


def _add_pos(rows_v, pos_v):
    @plsc.parallel_loop(0, SEQ, step=4)
    def _(r):
        for rr in range(4):
            for c in range(D // LANES):
                sl = pl.ds(c * LANES, LANES)
                plsc.addupdate(rows_v.at[r + rr, sl], pos_v[r + rr, sl])


def _sc_body(ids_hbm, pos_hbm, table_hbm, out_hbm,
             idx_v, rows0, rows1, rows2, pos_v,
             gsem0, gsem1, gsem2, wsem0, wsem1, wsem2):
    nc = 2
    wid = lax.axis_index("s") * nc + lax.axis_index("c")
    wbase = wid * CHUNKS * SEQ

    pltpu.sync_copy(pos_hbm, pos_v)
    pltpu.sync_copy(ids_hbm.at[pl.ds(wbase, CHUNKS * SEQ)], idx_v)

    rows = (rows0, rows1, rows2)
    gsem = (gsem0, gsem1, gsem2)
    wsem = (wsem0, wsem1, wsem2)

    def gather(g, p):
        pltpu.async_copy(table_hbm.at[idx_v.at[pl.ds(g * SEQ, SEQ)]],
                         rows[p], gsem[p])

    def gather_wait(p):
        pltpu.make_async_copy(table_hbm.at[idx_v.at[pl.ds(0, SEQ)]],
                              rows[p], gsem[p]).wait()

    def write(g, p):
        pltpu.async_copy(rows[p], out_hbm.at[pl.ds(wbase + g * SEQ, SEQ)],
                         wsem[p])

    def write_wait(p):
        pltpu.make_async_copy(rows[p], out_hbm.at[pl.ds(wbase, SEQ)],
                              wsem[p]).wait()

    def process(g, p):
        gather_wait(p)
        _add_pos(rows[p], pos_v)
        write(g, p)

    gather(0, 0)
    gather(1, 1)
    gather(2, 2)
    process(0, 0)
    process_delay = 2

    def triple_body(k, carry):
        for j in range(3):
            g = 3 * k + 3 + j
            write_wait(j)
            gather(g, j)
            pg = g - process_delay
            process(pg, (j + 1) % 3)
        return carry

    lax.fori_loop(0, (CHUNKS - 5) // 3, triple_body, 0)

    write_wait(0)
    gather(126, 0)
    process(124, 1)
    write_wait(1)
    gather(127, 1)
    process(125, 2)

    process(126, 0)
    process(127, 1)
    write_wait(2)
    write_wait(0)
    write_wait(1)


def kernel(task_token_ids, embed_table, pos):
    b, l = task_token_ids.shape
    ids_flat = task_token_ids.reshape(-1)
    pos2 = pos.reshape(pos.shape[1], pos.shape[2])[:l]

    mesh = plsc.VectorSubcoreMesh(core_axis_name="c", subcore_axis_name="s")
    run = functools.partial(
        pl.kernel,
        mesh=mesh,
        out_type=jax.ShapeDtypeStruct((b * l, D), jnp.float32),
        scratch_types=[
            pltpu.VMEM((CHUNKS * SEQ,), jnp.int32),
            pltpu.VMEM((SEQ, D), jnp.float32),
            pltpu.VMEM((SEQ, D), jnp.float32),
            pltpu.VMEM((SEQ, D), jnp.float32),
            pltpu.VMEM((SEQ, D), jnp.float32),
            pltpu.SemaphoreType.DMA,
            pltpu.SemaphoreType.DMA,
            pltpu.SemaphoreType.DMA,
            pltpu.SemaphoreType.DMA,
            pltpu.SemaphoreType.DMA,
            pltpu.SemaphoreType.DMA,
        ],
    )(_sc_body)

    out = run(ids_flat, pos2, embed_table)
    return out.reshape(b, l, D)


# speedup vs baseline: 1.1898x; 1.1898x over previous
import functools

import jax
import jax.numpy as jnp
from jax import lax
from jax.experimental import pallas as pl
from jax.experimental.pallas import tpu as pltpu
from jax.experimental.pallas import tpu_sc as plsc

D = 128
SEQ = 200
BATCH = 4096
LANES = 16
NWORKERS = 32
CHUNKS = BATCH // NWORKERS
NBUF = 4


def _make_ops(ids_hbm, table_hbm, out_hbm, wbase, ibuf, rows, pos_v,
              isem, gsem, wsem):
    def idx_issue(g, b):
        pltpu.async_copy(ids_hbm.at[pl.ds(wbase + g * SEQ, SEQ)],
                         ibuf[b], isem[b])

    def idx_wait(b):
        pltpu.make_async_copy(ids_hbm.at[pl.ds(0, SEQ)],
                              ibuf[b], isem[b]).wait()

    def gather(g, b):
        del g
        pltpu.async_copy(table_hbm.at[ibuf[b]], rows[b], gsem[b])

    def gather_wait(b):
        pltpu.make_async_copy(table_hbm.at[ibuf[b]], rows[b], gsem[b]).wait()

    def write(g, b):
        pltpu.async_copy(rows[b], out_hbm.at[pl.ds(wbase + g * SEQ, SEQ)],
                         wsem[b])

    def write_wait(b):
        pltpu.make_async_copy(rows[b], out_hbm.at[pl.ds(wbase, SEQ)],
                              wsem[b]).wait()

    def add(b):
        @plsc.parallel_loop(0, SEQ, step=4)
        def _(r):
            for rr in range(4):
                for c in range(D // LANES):
                    sl = pl.ds(c * LANES, LANES)
                    plsc.addupdate(rows[b].at[r + rr, sl],
                                   pos_v[r + rr, sl])

    return idx_issue, idx_wait, gather, gather_wait, write, write_wait, add


def _sc_body(ids_hbm, pos_hbm, table_hbm, out_hbm,
             i0, i1, i2, i3, r0, r1, r2, r3, pos_v,
             is0, is1, is2, is3, gs0, gs1, gs2, gs3,
             ws0, ws1, ws2, ws3):
    nc = 2
    wid = lax.axis_index("s") * nc + lax.axis_index("c")
    wbase = wid * CHUNKS * SEQ

    pltpu.sync_copy(pos_hbm, pos_v)

    (idx_issue, idx_wait, gather, gather_wait, write, write_wait, add
     ) = _make_ops(ids_hbm, table_hbm, out_hbm, wbase,
                   (i0, i1, i2, i3), (r0, r1, r2, r3), pos_v,
                   (is0, is1, is2, is3), (gs0, gs1, gs2, gs3),
                   (ws0, ws1, ws2, ws3))

    idx_issue(0, 0)
    idx_issue(1, 1)
    idx_wait(0); gather(0, 0); idx_issue(2, 2)
    idx_wait(1); gather(1, 1); idx_issue(3, 3)
    idx_wait(2); gather(2, 2); gather_wait(0); add(0); idx_issue(4, 0)
    idx_wait(3); gather(3, 3); write(0, 0)
    gather_wait(1); add(1); idx_issue(5, 1)

    def quad_body(k, carry):
        for j in range(4):
            g = 4 * k + 4 + j
            write_wait(j)
            idx_wait(j)
            gather(g, j)
            write(g - 3, (j + 1) % 4)
            gather_wait((j + 2) % 4)
            add((j + 2) % 4)
            idx_issue(g + 2, (j + 2) % 4)
        return carry

    lax.fori_loop(0, (CHUNKS - 8) // 4, quad_body, 0)

    write_wait(0); idx_wait(0); gather(124, 0); write(121, 1)
    gather_wait(2); add(2); idx_issue(126, 2)
    write_wait(1); idx_wait(1); gather(125, 1); write(122, 2)
    gather_wait(3); add(3); idx_issue(127, 3)
    write_wait(2); idx_wait(2); gather(126, 2); write(123, 3)
    gather_wait(0); add(0)
    write_wait(3); idx_wait(3); gather(127, 3); write(124, 0)
    gather_wait(1); add(1)

    write_wait(0); write(125, 1); gather_wait(2); add(2)
    write_wait(1); write(126, 2); gather_wait(3); add(3)
    write_wait(2); write(127, 3)
    write_wait(3)


def kernel(task_token_ids, embed_table, pos):
    b, l = task_token_ids.shape
    ids_flat = task_token_ids.reshape(-1)
    pos2 = pos.reshape(pos.shape[1], pos.shape[2])[:l]

    mesh = plsc.VectorSubcoreMesh(core_axis_name="c", subcore_axis_name="s")
    run = functools.partial(
        pl.kernel,
        mesh=mesh,
        out_type=jax.ShapeDtypeStruct((b * l, D), jnp.float32),
        scratch_types=(
            [pltpu.VMEM((SEQ,), jnp.int32) for _ in range(NBUF)]
            + [pltpu.VMEM((SEQ, D), jnp.float32) for _ in range(NBUF)]
            + [pltpu.VMEM((SEQ, D), jnp.float32)]
            + [pltpu.SemaphoreType.DMA for _ in range(3 * NBUF)]
        ),
    )(_sc_body)

    out = run(ids_flat, pos2, embed_table)
    return out.reshape(b, l, D)
